# baseline (speedup 1.0000x reference)
import functools

import jax
import jax.numpy as jnp
from jax import lax
from jax.experimental import pallas as pl
from jax.experimental.pallas import tpu as pltpu
from jax.experimental.pallas import tpu_sc as plsc

N = 8192
E = 262144
D = 128
ED = 32
NM = 64
NC = 2
NS = 16
NW = NC * NS
L = 16
EW = E // NW
CB = 128
NCHUNK = EW // CB
RPT = N // NS
SQRT_DIM = 1.0 / (128.0 ** 0.5)
EPS = 1e-8

_mesh = plsc.VectorSubcoreMesh(
    core_axis_name="c", subcore_axis_name="s", num_cores=NC, num_subcores=NS)


def _wid():
    return lax.axis_index("s") * NC + lax.axis_index("c")


def _sqrtv(x):
    yi = (plsc.bitcast(x, jnp.int32) >> 1) + 0x1FBD1DF5
    y = plsc.bitcast(yi, jnp.float32)
    for _ in range(4):
        y = 0.5 * (y + x / jnp.maximum(y, 1e-30))
    return y


def _globals_from_partials(pbuf, gvbuf):
    num = jnp.zeros((L,), jnp.float32)
    si2 = jnp.zeros((L,), jnp.float32)
    sj2 = jnp.zeros((L,), jnp.float32)
    mx = jnp.full((L,), -jnp.inf, jnp.float32)
    for w in range(NW):
        num = num + pbuf[pl.ds(w * 64, L)]
        si2 = si2 + pbuf[pl.ds(w * 64 + 16, L)]
        sj2 = sj2 + pbuf[pl.ds(w * 64 + 32, L)]
        mx = jnp.maximum(mx, pbuf[pl.ds(w * 64 + 48, L)])
    sim = num / (jnp.maximum(_sqrtv(si2), EPS) * jnp.maximum(_sqrtv(sj2), EPS))
    c = gvbuf[...] * sim
    return mx, c


@functools.partial(
    pl.kernel,
    out_type=(
        jax.ShapeDtypeStruct((E,), jnp.float32),
        jax.ShapeDtypeStruct((E,), jnp.float32),
        jax.ShapeDtypeStruct((NW * 64,), jnp.float32),
    ),
    mesh=_mesh,
    compiler_params=pltpu.CompilerParams(needs_layout_passes=False),
    scratch_types=[
        pltpu.VMEM((CB,), jnp.int32),
        pltpu.VMEM((CB,), jnp.int32),
        pltpu.VMEM((CB,), jnp.int32),
        pltpu.VMEM((CB, 256), jnp.float32),
        pltpu.VMEM((CB, 256), jnp.float32),
        pltpu.VMEM((CB,), jnp.float32),
        pltpu.VMEM((CB,), jnp.float32),
        pltpu.VMEM((L * L,), jnp.float32),
        pltpu.VMEM((N,), jnp.int32),
        pltpu.VMEM((NM,), jnp.float32),
        pltpu.VMEM((N,), jnp.float32),
        pltpu.VMEM((L,), jnp.float32),
        pltpu.VMEM((64,), jnp.float32),
        pltpu.SemaphoreType.DMA,
    ],
)
def _stage1(a_hbm, b_hbm, adj_hbm, src_hbm, dst_hbm, mids_hbm,
            mw_hbm, l0_hbm, s0_out, conn_out, part_out,
            srcb, dstb, fixb, qb, kb, adjb, s0b, tpb,
            midsb, mwb, embb, l0b, prow, sem):
    w = _wid()

    pltpu.sync_copy(mids_hbm, midsb)
    pltpu.sync_copy(mw_hbm, mwb)
    pltpu.sync_copy(l0_hbm, l0b)

    def emb_body(j, _):
        o = pl.multiple_of(j * L, L)
        embb[pl.ds(o, L)] = plsc.load_gather(mwb, [midsb[pl.ds(o, L)]])
        return 0

    lax.fori_loop(0, N // L, emb_body, 0)

    el0v = jnp.exp(l0b[...])
    lane = lax.broadcasted_iota(jnp.int32, (L,), 0)

    def chunk_body(i, carry):
        base = pl.multiple_of(w * EW + i * CB, CB)
        pltpu.sync_copy(src_hbm.at[pl.ds(base, CB)], srcb)
        pltpu.sync_copy(dst_hbm.at[pl.ds(base, CB)], dstb)
        for g in range(CB // L):
            sl = pl.ds(g * L, L)
            fixb[sl] = srcb[sl] * N + dstb[sl]
        hq = pltpu.async_copy(a_hbm.at[srcb], qb, sem)
        hk = pltpu.async_copy(b_hbm.at[dstb], kb, sem)
        ha = pltpu.async_copy(adj_hbm.at[fixb], adjb, sem)
        hq.wait(); hk.wait(); ha.wait()

        def group_body(g, carry):
            num_a, si2_a, sj2_a, max_a = carry
            for j in range(L):
                e = g * L + j
                ax = qb[e, pl.ds(0, L)] * kb[e, pl.ds(0, L)]
                for c in range(1, D // L):
                    ax = ax + qb[e, pl.ds(c * L, L)] * kb[e, pl.ds(c * L, L)]
                ay = qb[e, pl.ds(D, L)] * kb[e, pl.ds(D, L)]
                ay = ay + qb[e, pl.ds(D + L, L)] * kb[e, pl.ds(D + L, L)]
                tpb[pl.ds(j * L, L)] = ax * SQRT_DIM + ay * el0v
            s0v = plsc.load_gather(tpb, [lane * L])
            for l in range(1, L):
                s0v = s0v + plsc.load_gather(tpb, [lane * L + l])
            sl = pl.ds(pl.multiple_of(g * L, L), L)
            s0b[sl] = s0v
            es = plsc.load_gather(embb, [srcb[sl]])
            ed = plsc.load_gather(embb, [dstb[sl]])
            return (num_a + es * ed, si2_a + es * es, sj2_a + ed * ed,
                    jnp.maximum(max_a, s0v))

        carry = lax.fori_loop(0, CB // L, group_body, carry)
        pltpu.sync_copy(s0b, s0_out.at[pl.ds(base, CB)])
        pltpu.sync_copy(adjb, conn_out.at[pl.ds(base, CB)])
        return carry

    z = jnp.zeros((L,), jnp.float32)
    ninf = jnp.full((L,), -jnp.inf, jnp.float32)
    num_a, si2_a, sj2_a, max_a = lax.fori_loop(
        0, NCHUNK, chunk_body, (z, z, z, ninf))

    prow[pl.ds(0, L)] = jnp.full((L,), jnp.sum(num_a))
    prow[pl.ds(16, L)] = jnp.full((L,), jnp.sum(si2_a))
    prow[pl.ds(32, L)] = jnp.full((L,), jnp.sum(sj2_a))
    prow[pl.ds(48, L)] = jnp.full((L,), jnp.max(max_a))
    pltpu.sync_copy(prow, part_out.at[pl.ds(w * 64, 64)])


@functools.partial(
    pl.kernel,
    out_type=(
        jax.ShapeDtypeStruct((NC * N,), jnp.float32),
        jax.ShapeDtypeStruct((NC * N,), jnp.float32),
    ),
    mesh=_mesh,
    compiler_params=pltpu.CompilerParams(needs_layout_passes=False),
    scratch_types=[
        pltpu.VMEM((NW * 64,), jnp.float32),
        pltpu.VMEM((L,), jnp.float32),
        pltpu.VMEM((CB,), jnp.int32),
        pltpu.VMEM((CB,), jnp.float32),
        pltpu.VMEM((CB,), jnp.float32),
        pltpu.VMEM((CB,), jnp.float32),
        pltpu.VMEM((CB,), jnp.float32),
        pltpu.VMEM((RPT,), jnp.float32),
        pltpu.VMEM_SHARED((N,), jnp.float32),
        pltpu.VMEM_SHARED((N,), jnp.float32),
    ],
)
def _stage2(s0_hbm, conn_hbm, src_hbm, part_hbm, g_hbm,
            ssum0_out, ssum1_out,
            pbuf, gvb, srcb, s0b, cnb, e0b, e1b, zb, ss0_sp, ss1_sp):
    w = _wid()
    cid = lax.axis_index("c")
    sid = lax.axis_index("s")
    pltpu.sync_copy(part_hbm, pbuf)
    pltpu.sync_copy(g_hbm, gvb)
    mg, cc = _globals_from_partials(pbuf, gvb)

    for g in range(RPT // L):
        zb[pl.ds(g * L, L)] = jnp.zeros((L,), jnp.float32)
    rbase = pl.multiple_of(sid * RPT, RPT)
    pltpu.sync_copy(zb, ss0_sp.at[pl.ds(rbase, RPT)])
    pltpu.sync_copy(zb, ss1_sp.at[pl.ds(rbase, RPT)])
    plsc.subcore_barrier()

    def chunk_body(i, _):
        base = pl.multiple_of(w * EW + i * CB, CB)
        pltpu.sync_copy(src_hbm.at[pl.ds(base, CB)], srcb)
        pltpu.sync_copy(s0_hbm.at[pl.ds(base, CB)], s0b)
        pltpu.sync_copy(conn_hbm.at[pl.ds(base, CB)], cnb)
        for g in range(CB // L):
            sl = pl.ds(g * L, L)
            e0b[sl] = jnp.exp(s0b[sl] - mg)
            e1b[sl] = jnp.exp(cnb[sl] * cc)
        pltpu.sync_copy(e0b, ss0_sp.at[srcb], add=True)
        pltpu.sync_copy(e1b, ss1_sp.at[srcb], add=True)
        return 0

    lax.fori_loop(0, NCHUNK, chunk_body, 0)
    plsc.subcore_barrier()

    obase = pl.multiple_of(cid * N + sid * RPT, RPT)
    pltpu.sync_copy(ss0_sp.at[pl.ds(rbase, RPT)], zb)
    pltpu.sync_copy(zb, ssum0_out.at[pl.ds(obase, RPT)])
    pltpu.sync_copy(ss1_sp.at[pl.ds(rbase, RPT)], zb)
    pltpu.sync_copy(zb, ssum1_out.at[pl.ds(obase, RPT)])


@functools.partial(
    pl.kernel,
    out_type=jax.ShapeDtypeStruct((NC, N, D), jnp.float32),
    mesh=_mesh,
    compiler_params=pltpu.CompilerParams(needs_layout_passes=False),
    scratch_types=[
        pltpu.VMEM((NW * 64,), jnp.float32),
        pltpu.VMEM((L,), jnp.float32),
        pltpu.VMEM((512,), jnp.float32),
        pltpu.VMEM((N,), jnp.float32),
        pltpu.VMEM((N,), jnp.float32),
        pltpu.VMEM((CB,), jnp.int32),
        pltpu.VMEM((CB,), jnp.int32),
        pltpu.VMEM((CB,), jnp.float32),
        pltpu.VMEM((CB,), jnp.float32),
        pltpu.VMEM((CB,), jnp.float32),
        pltpu.VMEM((CB, D), jnp.float32),
        pltpu.VMEM((CB, D), jnp.float32),
        pltpu.VMEM_SHARED((N, D), jnp.float32),
        pltpu.SemaphoreType.DMA,
    ],
)
def _stage3(s0_hbm, conn_hbm, src_hbm, dst_hbm, v_hbm, part_hbm, g_hbm,
            ssum0_hbm, ssum1_hbm, out_hbm,
            pbuf, gvb, zb, ss0, ss1, srcb, dstb, s0b, cnb, wb,
            vrb, crb, acc_sp, sem):
    w = _wid()
    cid = lax.axis_index("c")
    sid = lax.axis_index("s")
    pltpu.sync_copy(part_hbm, pbuf)
    pltpu.sync_copy(g_hbm, gvb)
    mg, cc = _globals_from_partials(pbuf, gvb)

    pltpu.sync_copy(ssum0_hbm.at[pl.ds(0, N)], ss0)
    pltpu.sync_copy(ssum1_hbm.at[pl.ds(0, N)], ss1)
    for b in range(N // 512):
        pltpu.sync_copy(ssum0_hbm.at[pl.ds(N + b * 512, 512)], zb)
        for g in range(512 // L):
            sl = pl.ds(b * 512 + g * L, L)
            ss0[sl] = ss0[sl] + zb[pl.ds(g * L, L)]
    for b in range(N // 512):
        pltpu.sync_copy(ssum1_hbm.at[pl.ds(N + b * 512, 512)], zb)
        for g in range(512 // L):
            sl = pl.ds(b * 512 + g * L, L)
            ss1[sl] = ss1[sl] + zb[pl.ds(g * L, L)]

    def zero_body(e, _):
        for c in range(D // L):
            crb[e, pl.ds(c * L, L)] = jnp.zeros((L,), jnp.float32)
        return 0

    lax.fori_loop(0, CB, zero_body, 0)
    rbase = pl.multiple_of(sid * RPT, RPT)
    for j in range(RPT // CB):
        pltpu.sync_copy(crb, acc_sp.at[pl.ds(rbase + j * CB, CB)])
    plsc.subcore_barrier()

    def chunk_body(i, _):
        base = pl.multiple_of(w * EW + i * CB, CB)
        pltpu.sync_copy(src_hbm.at[pl.ds(base, CB)], srcb)
        pltpu.sync_copy(dst_hbm.at[pl.ds(base, CB)], dstb)
        pltpu.sync_copy(s0_hbm.at[pl.ds(base, CB)], s0b)
        pltpu.sync_copy(conn_hbm.at[pl.ds(base, CB)], cnb)
        hv = pltpu.async_copy(v_hbm.at[dstb], vrb, sem)
        for g in range(CB // L):
            sl = pl.ds(g * L, L)
            ev0 = jnp.exp(s0b[sl] - mg)
            ev1 = jnp.exp(cnb[sl] * cc)
            sg0 = plsc.load_gather(ss0, [srcb[sl]])
            sg1 = plsc.load_gather(ss1, [srcb[sl]])
            wb[sl] = 0.5 * (ev0 / sg0 + ev1 / sg1)
        hv.wait()

        def scale_body(e, _):
            wv = plsc.load_gather(wb, [jnp.full((L,), e, jnp.int32)])
            for c in range(D // L):
                crb[e, pl.ds(c * L, L)] = vrb[e, pl.ds(c * L, L)] * wv
            return 0

        lax.fori_loop(0, CB, scale_body, 0)
        pltpu.sync_copy(crb, acc_sp.at[srcb], add=True)
        return 0

    lax.fori_loop(0, NCHUNK, chunk_body, 0)
    plsc.subcore_barrier()

    for j in range(RPT // CB):
        pltpu.sync_copy(acc_sp.at[pl.ds(rbase + j * CB, CB)], crb)
        pltpu.sync_copy(crb, out_hbm.at[cid, pl.ds(rbase + j * CB, CB)])


def _tables_body(q_ref, k_ref, e_ref, a_ref, b_ref):
    a_ref[:, :D] = q_ref[...]
    a_ref[:, D:D + ED] = e_ref[...]
    b_ref[:, :D] = k_ref[...]
    b_ref[:, D:D + ED] = e_ref[...]


def _tables(q, k, eigs):
    blk = 512
    return pl.pallas_call(
        _tables_body,
        grid=(N // blk,),
        in_specs=[
            pl.BlockSpec((blk, D), lambda i: (i, 0)),
            pl.BlockSpec((blk, D), lambda i: (i, 0)),
            pl.BlockSpec((blk, ED), lambda i: (i, 0)),
        ],
        out_specs=[
            pl.BlockSpec((blk, 256), lambda i: (i, 0)),
            pl.BlockSpec((blk, 256), lambda i: (i, 0)),
        ],
        out_shape=[
            jax.ShapeDtypeStruct((N, 256), jnp.float32),
            jax.ShapeDtypeStruct((N, 256), jnp.float32),
        ],
    )(q, k, eigs)


def _combine_body(a_ref, b_ref, o_ref):
    o_ref[...] = a_ref[0] + b_ref[0]


def _combine(parts):
    return pl.pallas_call(
        _combine_body,
        grid=(16,),
        in_specs=[
            pl.BlockSpec((1, N // 16, D), lambda i: (0, i, 0)),
            pl.BlockSpec((1, N // 16, D), lambda i: (1, i, 0)),
        ],
        out_specs=pl.BlockSpec((N // 16, D), lambda i: (i, 0)),
        out_shape=jax.ShapeDtypeStruct((N, D), jnp.float32),
    )(parts, parts)


def kernel(q, k, v, indices, eigs, motif_Adj, motif_ids, lambda0, gamma, motif_w):
    src = indices[0].astype(jnp.int32)
    dst = indices[1].astype(jnp.int32)
    adj_flat = motif_Adj.reshape(N * N)
    mids = motif_ids.astype(jnp.int32)
    mw = motif_w.reshape(NM)
    l0v = jnp.broadcast_to(lambda0.astype(jnp.float32), (L,))
    gv = jnp.broadcast_to(gamma.astype(jnp.float32), (L,))

    a_tab, b_tab = _tables(q, k, eigs)
    s0_all, conn_all, partials = _stage1(
        a_tab, b_tab, adj_flat, src, dst, mids, mw, l0v)
    ssum0, ssum1 = _stage2(s0_all, conn_all, src, partials, gv)
    parts = _stage3(s0_all, conn_all, src, dst, v, partials, gv, ssum0, ssum1)
    return _combine(parts)


# speedup vs baseline: 1.0466x; 1.0466x over previous
import functools

import jax
import jax.numpy as jnp
from jax import lax
from jax.experimental import pallas as pl
from jax.experimental.pallas import tpu as pltpu
from jax.experimental.pallas import tpu_sc as plsc

N = 8192
E = 262144
D = 128
ED = 32
NM = 64
NC = 2
NS = 16
NW = NC * NS
L = 16
EW = E // NW
CB = 128
NCHUNK = EW // CB
RPT = N // NS
SQRT_DIM = 1.0 / (128.0 ** 0.5)
EPS = 1e-8

_mesh = plsc.VectorSubcoreMesh(
    core_axis_name="c", subcore_axis_name="s", num_cores=NC, num_subcores=NS)


def _wid():
    return lax.axis_index("s") * NC + lax.axis_index("c")


def _sqrtv(x):
    yi = (plsc.bitcast(x, jnp.int32) >> 1) + 0x1FBD1DF5
    y = plsc.bitcast(yi, jnp.float32)
    for _ in range(4):
        y = 0.5 * (y + x / jnp.maximum(y, 1e-30))
    return y


def _globals_from_partials(pbuf, gvbuf):
    num = jnp.zeros((L,), jnp.float32)
    si2 = jnp.zeros((L,), jnp.float32)
    sj2 = jnp.zeros((L,), jnp.float32)
    mx = jnp.full((L,), -jnp.inf, jnp.float32)
    for w in range(NW):
        num = num + pbuf[pl.ds(w * 64, L)]
        si2 = si2 + pbuf[pl.ds(w * 64 + 16, L)]
        sj2 = sj2 + pbuf[pl.ds(w * 64 + 32, L)]
        mx = jnp.maximum(mx, pbuf[pl.ds(w * 64 + 48, L)])
    sim = num / (jnp.maximum(_sqrtv(si2), EPS) * jnp.maximum(_sqrtv(sj2), EPS))
    c = gvbuf[...] * sim
    return mx, c


CB1 = 64
NCHUNK1 = EW // CB1


@functools.partial(
    pl.kernel,
    out_type=(
        jax.ShapeDtypeStruct((E,), jnp.float32),
        jax.ShapeDtypeStruct((E,), jnp.float32),
        jax.ShapeDtypeStruct((NW * 64,), jnp.float32),
    ),
    mesh=_mesh,
    compiler_params=pltpu.CompilerParams(needs_layout_passes=False),
    scratch_types=[
        pltpu.VMEM((2, CB1), jnp.int32),
        pltpu.VMEM((2, CB1), jnp.int32),
        pltpu.VMEM((2, CB1), jnp.int32),
        pltpu.VMEM((2, CB1, 256), jnp.float32),
        pltpu.VMEM((2, CB1, 256), jnp.float32),
        pltpu.VMEM((2, CB1), jnp.float32),
        pltpu.VMEM((CB1,), jnp.float32),
        pltpu.VMEM((L * L,), jnp.float32),
        pltpu.VMEM((N,), jnp.int32),
        pltpu.VMEM((NM,), jnp.float32),
        pltpu.VMEM((N,), jnp.float32),
        pltpu.VMEM((L,), jnp.float32),
        pltpu.VMEM((64,), jnp.float32),
        pltpu.SemaphoreType.DMA,
        pltpu.SemaphoreType.DMA,
        pltpu.SemaphoreType.DMA,
        pltpu.SemaphoreType.DMA,
    ],
)
def _stage1(a_hbm, b_hbm, adj_hbm, src_hbm, dst_hbm, mids_hbm,
            mw_hbm, l0_hbm, s0_out, conn_out, part_out,
            srcb, dstb, fixb, qb, kb, adjb, s0b, tpb,
            midsb, mwb, embb, l0b, prow, semi0, semi1, semr0, semr1):
    w = _wid()
    semi = (semi0, semi1)
    semr = (semr0, semr1)

    pltpu.sync_copy(mids_hbm, midsb)
    pltpu.sync_copy(mw_hbm, mwb)
    pltpu.sync_copy(l0_hbm, l0b)

    def emb_body(j, _):
        o = pl.multiple_of(j * L, L)
        embb[pl.ds(o, L)] = plsc.load_gather(mwb, [midsb[pl.ds(o, L)]])
        return 0

    lax.fori_loop(0, N // L, emb_body, 0)

    el0v = jnp.exp(l0b[...])
    lane = lax.broadcasted_iota(jnp.int32, (L,), 0)

    def issue_idx(i, b):
        base = pl.multiple_of(w * EW + i * CB1, CB1)
        pltpu.async_copy(src_hbm.at[pl.ds(base, CB1)], srcb.at[b], semi[b])
        pltpu.async_copy(dst_hbm.at[pl.ds(base, CB1)], dstb.at[b], semi[b])

    def drain_idx(i, b):
        base = pl.multiple_of(w * EW + i * CB1, CB1)
        pltpu.make_async_copy(src_hbm.at[pl.ds(base, CB1)], srcb.at[b], semi[b]).wait()
        pltpu.make_async_copy(dst_hbm.at[pl.ds(base, CB1)], dstb.at[b], semi[b]).wait()

    def fidx_and_issue_rows(b):
        for g in range(CB1 // L):
            sl = pl.ds(g * L, L)
            fixb[b, sl] = srcb[b, sl] * N + dstb[b, sl]
        pltpu.async_copy(a_hbm.at[srcb.at[b]], qb.at[b], semr[b])
        pltpu.async_copy(b_hbm.at[dstb.at[b]], kb.at[b], semr[b])
        pltpu.async_copy(adj_hbm.at[fixb.at[b]], adjb.at[b], semr[b])

    def drain_rows(b):
        pltpu.make_async_copy(a_hbm.at[srcb.at[b]], qb.at[b], semr[b]).wait()
        pltpu.make_async_copy(b_hbm.at[dstb.at[b]], kb.at[b], semr[b]).wait()
        pltpu.make_async_copy(adj_hbm.at[fixb.at[b]], adjb.at[b], semr[b]).wait()

    issue_idx(0, 0)
    drain_idx(0, 0)
    fidx_and_issue_rows(0)

    def pair_body(gp, carry):
        for b in range(2):
            i = gp * 2 + b
            nb = 1 - b
            drain_rows(b)

            @pl.when(i + 1 < NCHUNK1)
            def _():
                issue_idx(i + 1, nb)

            def group_body(g, carry):
                num_a, si2_a, sj2_a, max_a = carry
                for j in range(L):
                    e = g * L + j
                    ax = qb[b, e, pl.ds(0, L)] * kb[b, e, pl.ds(0, L)]
                    for c in range(1, D // L):
                        ax = ax + qb[b, e, pl.ds(c * L, L)] * kb[b, e, pl.ds(c * L, L)]
                    ay = qb[b, e, pl.ds(D, L)] * kb[b, e, pl.ds(D, L)]
                    ay = ay + qb[b, e, pl.ds(D + L, L)] * kb[b, e, pl.ds(D + L, L)]
                    tpb[pl.ds(j * L, L)] = ax * SQRT_DIM + ay * el0v
                s0v = plsc.load_gather(tpb, [lane * L])
                for l in range(1, L):
                    s0v = s0v + plsc.load_gather(tpb, [lane * L + l])
                sl = pl.ds(pl.multiple_of(g * L, L), L)
                s0b[sl] = s0v
                es = plsc.load_gather(embb, [srcb[b, sl]])
                ed = plsc.load_gather(embb, [dstb[b, sl]])
                return (num_a + es * ed, si2_a + es * es, sj2_a + ed * ed,
                        jnp.maximum(max_a, s0v))

            carry = lax.fori_loop(0, CB1 // L, group_body, carry)
            base = pl.multiple_of(w * EW + i * CB1, CB1)
            pltpu.sync_copy(s0b, s0_out.at[pl.ds(base, CB1)])
            pltpu.sync_copy(adjb.at[b], conn_out.at[pl.ds(base, CB1)])

            @pl.when(i + 1 < NCHUNK1)
            def _():
                drain_idx(i + 1, nb)
                fidx_and_issue_rows(nb)
        return carry

    z = jnp.zeros((L,), jnp.float32)
    ninf = jnp.full((L,), -jnp.inf, jnp.float32)
    num_a, si2_a, sj2_a, max_a = lax.fori_loop(
        0, NCHUNK1 // 2, pair_body, (z, z, z, ninf))

    prow[pl.ds(0, L)] = jnp.full((L,), jnp.sum(num_a))
    prow[pl.ds(16, L)] = jnp.full((L,), jnp.sum(si2_a))
    prow[pl.ds(32, L)] = jnp.full((L,), jnp.sum(sj2_a))
    prow[pl.ds(48, L)] = jnp.full((L,), jnp.max(max_a))
    pltpu.sync_copy(prow, part_out.at[pl.ds(w * 64, 64)])


@functools.partial(
    pl.kernel,
    out_type=(
        jax.ShapeDtypeStruct((NC * N,), jnp.float32),
        jax.ShapeDtypeStruct((NC * N,), jnp.float32),
    ),
    mesh=_mesh,
    compiler_params=pltpu.CompilerParams(needs_layout_passes=False),
    scratch_types=[
        pltpu.VMEM((NW * 64,), jnp.float32),
        pltpu.VMEM((L,), jnp.float32),
        pltpu.VMEM((CB,), jnp.int32),
        pltpu.VMEM((CB,), jnp.float32),
        pltpu.VMEM((CB,), jnp.float32),
        pltpu.VMEM((CB,), jnp.float32),
        pltpu.VMEM((CB,), jnp.float32),
        pltpu.VMEM((RPT,), jnp.float32),
        pltpu.VMEM_SHARED((N,), jnp.float32),
        pltpu.VMEM_SHARED((N,), jnp.float32),
    ],
)
def _stage2(s0_hbm, conn_hbm, src_hbm, part_hbm, g_hbm,
            ssum0_out, ssum1_out,
            pbuf, gvb, srcb, s0b, cnb, e0b, e1b, zb, ss0_sp, ss1_sp):
    w = _wid()
    cid = lax.axis_index("c")
    sid = lax.axis_index("s")
    pltpu.sync_copy(part_hbm, pbuf)
    pltpu.sync_copy(g_hbm, gvb)
    mg, cc = _globals_from_partials(pbuf, gvb)

    for g in range(RPT // L):
        zb[pl.ds(g * L, L)] = jnp.zeros((L,), jnp.float32)
    rbase = pl.multiple_of(sid * RPT, RPT)
    pltpu.sync_copy(zb, ss0_sp.at[pl.ds(rbase, RPT)])
    pltpu.sync_copy(zb, ss1_sp.at[pl.ds(rbase, RPT)])
    plsc.subcore_barrier()

    def chunk_body(i, _):
        base = pl.multiple_of(w * EW + i * CB, CB)
        pltpu.sync_copy(src_hbm.at[pl.ds(base, CB)], srcb)
        pltpu.sync_copy(s0_hbm.at[pl.ds(base, CB)], s0b)
        pltpu.sync_copy(conn_hbm.at[pl.ds(base, CB)], cnb)
        for g in range(CB // L):
            sl = pl.ds(g * L, L)
            e0b[sl] = jnp.exp(s0b[sl] - mg)
            e1b[sl] = jnp.exp(cnb[sl] * cc)
        pltpu.sync_copy(e0b, ss0_sp.at[srcb], add=True)
        pltpu.sync_copy(e1b, ss1_sp.at[srcb], add=True)
        return 0

    lax.fori_loop(0, NCHUNK, chunk_body, 0)
    plsc.subcore_barrier()

    obase = pl.multiple_of(cid * N + sid * RPT, RPT)
    pltpu.sync_copy(ss0_sp.at[pl.ds(rbase, RPT)], zb)
    pltpu.sync_copy(zb, ssum0_out.at[pl.ds(obase, RPT)])
    pltpu.sync_copy(ss1_sp.at[pl.ds(rbase, RPT)], zb)
    pltpu.sync_copy(zb, ssum1_out.at[pl.ds(obase, RPT)])


CB3 = 64
NCHUNK3 = EW // CB3
@functools.partial(
    pl.kernel,
    out_type=jax.ShapeDtypeStruct((NC, N, D), jnp.float32),
    mesh=_mesh,
    compiler_params=pltpu.CompilerParams(needs_layout_passes=False),
    scratch_types=[
        pltpu.VMEM((NW * 64,), jnp.float32),
        pltpu.VMEM((L,), jnp.float32),
        pltpu.VMEM((512,), jnp.float32),
        pltpu.VMEM((N,), jnp.float32),
        pltpu.VMEM((N,), jnp.float32),
        pltpu.VMEM((2, CB3), jnp.int32),
        pltpu.VMEM((2, CB3), jnp.int32),
        pltpu.VMEM((2, CB3), jnp.float32),
        pltpu.VMEM((2, CB3), jnp.float32),
        pltpu.VMEM((CB3,), jnp.float32),
        pltpu.VMEM((2, CB3, D), jnp.float32),
        pltpu.VMEM((CB3, D), jnp.float32),
        pltpu.VMEM_SHARED((N, D), jnp.float32),
        pltpu.SemaphoreType.DMA,
        pltpu.SemaphoreType.DMA,
        pltpu.SemaphoreType.DMA,
        pltpu.SemaphoreType.DMA,
    ],
)
def _stage3(s0_hbm, conn_hbm, src_hbm, dst_hbm, v_hbm, part_hbm, g_hbm,
            ssum0_hbm, ssum1_hbm, out_hbm,
            pbuf, gvb, zb, ss0, ss1, srcb, dstb, s0b, cnb, wb,
            vrb, crb, acc_sp, semi0, semi1, semr0, semr1):
    w = _wid()
    cid = lax.axis_index("c")
    sid = lax.axis_index("s")
    semi = (semi0, semi1)
    semr = (semr0, semr1)
    pltpu.sync_copy(part_hbm, pbuf)
    pltpu.sync_copy(g_hbm, gvb)
    mg, cc = _globals_from_partials(pbuf, gvb)

    pltpu.sync_copy(ssum0_hbm.at[pl.ds(0, N)], ss0)
    pltpu.sync_copy(ssum1_hbm.at[pl.ds(0, N)], ss1)
    for b in range(N // 512):
        pltpu.sync_copy(ssum0_hbm.at[pl.ds(N + b * 512, 512)], zb)
        for g in range(512 // L):
            sl = pl.ds(b * 512 + g * L, L)
            ss0[sl] = ss0[sl] + zb[pl.ds(g * L, L)]
    for b in range(N // 512):
        pltpu.sync_copy(ssum1_hbm.at[pl.ds(N + b * 512, 512)], zb)
        for g in range(512 // L):
            sl = pl.ds(b * 512 + g * L, L)
            ss1[sl] = ss1[sl] + zb[pl.ds(g * L, L)]

    def zero_body(e, _):
        for c in range(D // L):
            crb[e, pl.ds(c * L, L)] = jnp.zeros((L,), jnp.float32)
        return 0

    lax.fori_loop(0, CB3, zero_body, 0)
    rbase = pl.multiple_of(sid * RPT, RPT)
    for j in range(RPT // CB3):
        pltpu.sync_copy(crb, acc_sp.at[pl.ds(rbase + j * CB3, CB3)])
    plsc.subcore_barrier()

    def issue_lin(i, b):
        base = pl.multiple_of(w * EW + i * CB3, CB3)
        pltpu.async_copy(src_hbm.at[pl.ds(base, CB3)], srcb.at[b], semi[b])
        pltpu.async_copy(dst_hbm.at[pl.ds(base, CB3)], dstb.at[b], semi[b])
        pltpu.async_copy(s0_hbm.at[pl.ds(base, CB3)], s0b.at[b], semi[b])
        pltpu.async_copy(conn_hbm.at[pl.ds(base, CB3)], cnb.at[b], semi[b])

    def drain_lin(i, b):
        base = pl.multiple_of(w * EW + i * CB3, CB3)
        pltpu.make_async_copy(src_hbm.at[pl.ds(base, CB3)], srcb.at[b], semi[b]).wait()
        pltpu.make_async_copy(dst_hbm.at[pl.ds(base, CB3)], dstb.at[b], semi[b]).wait()
        pltpu.make_async_copy(s0_hbm.at[pl.ds(base, CB3)], s0b.at[b], semi[b]).wait()
        pltpu.make_async_copy(conn_hbm.at[pl.ds(base, CB3)], cnb.at[b], semi[b]).wait()

    def issue_rows(b):
        pltpu.async_copy(v_hbm.at[dstb.at[b]], vrb.at[b], semr[b])

    def drain_rows(b):
        pltpu.make_async_copy(v_hbm.at[dstb.at[b]], vrb.at[b], semr[b]).wait()

    issue_lin(0, 0)
    drain_lin(0, 0)
    issue_rows(0)

    def pair_body(gp, _):
        for b in range(2):
            i = gp * 2 + b
            nb = 1 - b

            @pl.when(i + 1 < NCHUNK3)
            def _():
                issue_lin(i + 1, nb)

            drain_rows(b)
            for g in range(CB3 // L):
                sl = pl.ds(g * L, L)
                ev0 = jnp.exp(s0b[b, sl] - mg)
                ev1 = jnp.exp(cnb[b, sl] * cc)
                sg0 = plsc.load_gather(ss0, [srcb[b, sl]])
                sg1 = plsc.load_gather(ss1, [srcb[b, sl]])
                wb[sl] = 0.5 * (ev0 / sg0 + ev1 / sg1)

            def scale_body(e, _):
                wv = plsc.load_gather(wb, [jnp.full((L,), e, jnp.int32)])
                for c in range(D // L):
                    crb[e, pl.ds(c * L, L)] = vrb[b, e, pl.ds(c * L, L)] * wv
                return 0

            lax.fori_loop(0, CB3, scale_body, 0)
            pltpu.sync_copy(crb, acc_sp.at[srcb.at[b]], add=True)

            @pl.when(i + 1 < NCHUNK3)
            def _():
                drain_lin(i + 1, nb)
                issue_rows(nb)
        return 0

    lax.fori_loop(0, NCHUNK3 // 2, pair_body, 0)
    plsc.subcore_barrier()

    for j in range(RPT // CB3):
        pltpu.sync_copy(acc_sp.at[pl.ds(rbase + j * CB3, CB3)], crb)
        pltpu.sync_copy(crb, out_hbm.at[cid, pl.ds(rbase + j * CB3, CB3)])


def _tables_body(q_ref, k_ref, e_ref, a_ref, b_ref):
    a_ref[:, :D] = q_ref[...]
    a_ref[:, D:D + ED] = e_ref[...]
    b_ref[:, :D] = k_ref[...]
    b_ref[:, D:D + ED] = e_ref[...]


def _tables(q, k, eigs):
    blk = 512
    return pl.pallas_call(
        _tables_body,
        grid=(N // blk,),
        in_specs=[
            pl.BlockSpec((blk, D), lambda i: (i, 0)),
            pl.BlockSpec((blk, D), lambda i: (i, 0)),
            pl.BlockSpec((blk, ED), lambda i: (i, 0)),
        ],
        out_specs=[
            pl.BlockSpec((blk, 256), lambda i: (i, 0)),
            pl.BlockSpec((blk, 256), lambda i: (i, 0)),
        ],
        out_shape=[
            jax.ShapeDtypeStruct((N, 256), jnp.float32),
            jax.ShapeDtypeStruct((N, 256), jnp.float32),
        ],
    )(q, k, eigs)


def _combine_body(a_ref, b_ref, o_ref):
    o_ref[...] = a_ref[0] + b_ref[0]


def _combine(parts):
    return pl.pallas_call(
        _combine_body,
        grid=(16,),
        in_specs=[
            pl.BlockSpec((1, N // 16, D), lambda i: (0, i, 0)),
            pl.BlockSpec((1, N // 16, D), lambda i: (1, i, 0)),
        ],
        out_specs=pl.BlockSpec((N // 16, D), lambda i: (i, 0)),
        out_shape=jax.ShapeDtypeStruct((N, D), jnp.float32),
    )(parts, parts)


def kernel(q, k, v, indices, eigs, motif_Adj, motif_ids, lambda0, gamma, motif_w):
    src = indices[0].astype(jnp.int32)
    dst = indices[1].astype(jnp.int32)
    adj_flat = motif_Adj.reshape(N * N)
    mids = motif_ids.astype(jnp.int32)
    mw = motif_w.reshape(NM)
    l0v = jnp.broadcast_to(lambda0.astype(jnp.float32), (L,))
    gv = jnp.broadcast_to(gamma.astype(jnp.float32), (L,))

    a_tab, b_tab = _tables(q, k, eigs)
    s0_all, conn_all, partials = _stage1(
        a_tab, b_tab, adj_flat, src, dst, mids, mw, l0v)
    ssum0, ssum1 = _stage2(s0_all, conn_all, src, partials, gv)
    parts = _stage3(s0_all, conn_all, src, dst, v, partials, gv, ssum0, ssum1)
    return _combine(parts)
